# Initial kernel scaffold; baseline (speedup 1.0000x reference)
import jax
import jax.numpy as jnp
from jax.experimental import pallas as pl


def kernel(coord, edge_index, targets, node_W, node_b, msg_W, msg_b, upd_W, upd_b, edge_W, edge_b, yhat_W, yhat_b):
    raise NotImplementedError("write your pallas kernel here")


# speedup vs baseline: 1.0375x; 1.0375x over previous
import functools

import jax
import jax.numpy as jnp
from jax.experimental import pallas as pl

N = 50000
E = 800000
D = 64
L = 3
BR = 2000


def _upd_body(x_ref, agg_ref, w1_ref, w2_ref, b_ref, o_ref):
    h = (x_ref[...] @ w1_ref[...] + agg_ref[...] @ w2_ref[...]) + b_ref[...]
    o_ref[...] = jnp.maximum(h, 0.0) + x_ref[...]


def _upd_layer(x, agg, w1, w2, b):
    return pl.pallas_call(
        _upd_body,
        out_shape=jax.ShapeDtypeStruct((N, D), jnp.float32),
        grid=(N // BR,),
        in_specs=[
            pl.BlockSpec((BR, D), lambda i: (i, 0)),
            pl.BlockSpec((BR, D), lambda i: (i, 0)),
            pl.BlockSpec((D, D), lambda i: (0, 0)),
            pl.BlockSpec((D, D), lambda i: (0, 0)),
            pl.BlockSpec((1, D), lambda i: (0, 0)),
        ],
        out_specs=pl.BlockSpec((BR, D), lambda i: (i, 0)),
    )(x, agg, w1, w2, b.reshape(1, D))


def kernel(coord, edge_index, targets, node_W, node_b, msg_W, msg_b,
           upd_W, upd_b, edge_W, edge_b, yhat_W, yhat_b):
    src = edge_index[0]
    dst = edge_index[1]
    x = coord @ node_W + node_b
    for l in range(L):
        xw = x @ msg_W[l] + msg_b[l]
        m = xw[src]
        agg = jax.ops.segment_min(m, dst, num_segments=N)
        agg = jnp.where(jnp.isfinite(agg), agg, 0.0)
        x = _upd_layer(x, agg, upd_W[l][:D], upd_W[l][D:], upd_b[l])
    xa = x @ edge_W[:D]
    xb = x @ edge_W[D:]
    ee = jax.nn.relu(xa[src] + xb[dst] + edge_b)
    y = (ee @ yhat_W + yhat_b).reshape(-1)
    return jnp.mean(jnp.abs(y - targets))
